# baseline (speedup 1.0000x reference)
import jax
import jax.numpy as jnp
from jax.experimental import pallas as pl
from jax.experimental.pallas import tpu as pltpu


def _make_fused_kernel(C, T, V, K):
    def _fused_kernel(x_ref, a_ref, w_ref, shift_ref, o_ref):
        O = w_ref.shape[0]
        xe = x_ref[0, :, 0::2, :].reshape(C * T // 2, V)
        xo = x_ref[0, :, 1::2, :].reshape(C * T // 2, V)
        xp = jnp.concatenate([xe, xo], axis=1).astype(jnp.bfloat16)
        cat = jnp.dot(xp, a_ref[...],
                      preferred_element_type=jnp.float32).astype(jnp.bfloat16)
        parts = [cat[:, 2 * V * k:2 * V * (k + 1)].reshape(C, T * V)
                 for k in range(K)]
        parts.append(xp.reshape(C, T * V))
        big = jnp.concatenate(parts, axis=0)
        y = jnp.dot(w_ref[...], big, preferred_element_type=jnp.float32)
        y = jnp.maximum(y + shift_ref[...], 0.0)
        o_ref[0] = y.reshape(O, T, V)
    return _fused_kernel


@jax.jit
def kernel(x_nctv, A, W, b, bn_gamma, bn_beta, bn_mean, bn_var,
           Wd, bd, dbn_gamma, dbn_beta, dbn_mean, dbn_var, eps=1e-5):
    N, C, T, V = x_nctv.shape
    K, O, _ = W.shape

    An = A / (jnp.sqrt(jnp.sum(A * A, axis=1, keepdims=True)) + 1e-4)
    bn_scale = bn_gamma / jnp.sqrt(bn_var + eps)
    bn_shift = bn_beta - bn_mean * bn_scale
    d_scale = dbn_gamma / jnp.sqrt(dbn_var + eps)
    d_shift = dbn_beta - dbn_mean * d_scale

    W_fold = W * bn_scale[None, :, None]
    Wd_fold = Wd * d_scale[:, None]
    W_all = jnp.concatenate([W_fold[0], W_fold[1], W_fold[2], Wd_fold],
                            axis=1).astype(jnp.bfloat16)
    shift = (bn_scale * jnp.sum(b, axis=0) + bn_shift
             + d_scale * bd + d_shift).reshape(O, 1)

    z = jnp.zeros((K, V, V), jnp.float32)
    A2 = jnp.concatenate([jnp.concatenate([An, z], axis=2),
                          jnp.concatenate([z, An], axis=2)], axis=1)
    A2cat = jnp.concatenate([A2[k] for k in range(K)],
                            axis=1).astype(jnp.bfloat16)

    out = pl.pallas_call(
        _make_fused_kernel(C, T, V, K),
        out_shape=jax.ShapeDtypeStruct((N, O, T, V), jnp.float32),
        grid=(N,),
        in_specs=[
            pl.BlockSpec((1, C, T, V), lambda n: (n, 0, 0, 0)),
            pl.BlockSpec((2 * V, K * 2 * V), lambda n: (0, 0)),
            pl.BlockSpec((O, (K + 1) * C), lambda n: (0, 0)),
            pl.BlockSpec((O, 1), lambda n: (0, 0)),
        ],
        out_specs=pl.BlockSpec((1, O, T, V), lambda n: (n, 0, 0, 0)),
        compiler_params=pltpu.CompilerParams(
            dimension_semantics=("parallel",)),
    )(x_nctv, A2cat, W_all, shift)

    return out


# speedup vs baseline: 2.6458x; 2.6458x over previous
import jax
import jax.numpy as jnp
from jax.experimental import pallas as pl
from jax.experimental.pallas import tpu as pltpu


def _make_fused_kernel(C, T, V, K):
    def _fused_kernel(x_ref, a_ref, w_ref, shift_ref, o_ref):
        O = w_ref.shape[0]
        xe = x_ref[0, :, 0::2, :].reshape(C * T // 2, V)
        xo = x_ref[0, :, 1::2, :].reshape(C * T // 2, V)
        xp = jnp.concatenate([xe, xo], axis=1).astype(jnp.bfloat16)
        cat = jnp.dot(xp, a_ref[...],
                      preferred_element_type=jnp.float32).astype(jnp.bfloat16)
        parts = [cat[:, 2 * V * k:2 * V * (k + 1)].reshape(C, T * V)
                 for k in range(K)]
        parts.append(xp.reshape(C, T * V))
        big = jnp.concatenate(parts, axis=0)
        yT = jax.lax.dot_general(big, w_ref[...], (((0,), (1,)), ((), ())),
                                 preferred_element_type=jnp.float32)
        yT = jnp.maximum(yT + shift_ref[...], 0.0)
        o_ref[0] = yT.reshape(T, V, O)
    return _fused_kernel


@jax.jit
def kernel(x_nctv, A, W, b, bn_gamma, bn_beta, bn_mean, bn_var,
           Wd, bd, dbn_gamma, dbn_beta, dbn_mean, dbn_var, eps=1e-5):
    N, C, T, V = x_nctv.shape
    K, O, _ = W.shape

    An = A / (jnp.sqrt(jnp.sum(A * A, axis=1, keepdims=True)) + 1e-4)
    bn_scale = bn_gamma / jnp.sqrt(bn_var + eps)
    bn_shift = bn_beta - bn_mean * bn_scale
    d_scale = dbn_gamma / jnp.sqrt(dbn_var + eps)
    d_shift = dbn_beta - dbn_mean * d_scale

    W_fold = W * bn_scale[None, :, None]
    Wd_fold = Wd * d_scale[:, None]
    W_all = jnp.concatenate([W_fold[0], W_fold[1], W_fold[2], Wd_fold],
                            axis=1).astype(jnp.bfloat16)
    shift = (bn_scale * jnp.sum(b, axis=0) + bn_shift
             + d_scale * bd + d_shift).reshape(1, O)

    z = jnp.zeros((K, V, V), jnp.float32)
    A2 = jnp.concatenate([jnp.concatenate([An, z], axis=2),
                          jnp.concatenate([z, An], axis=2)], axis=1)
    A2cat = jnp.concatenate([A2[k] for k in range(K)],
                            axis=1).astype(jnp.bfloat16)

    out_tvo = pl.pallas_call(
        _make_fused_kernel(C, T, V, K),
        out_shape=jax.ShapeDtypeStruct((N, T, V, O), jnp.float32),
        grid=(N,),
        in_specs=[
            pl.BlockSpec((1, C, T, V), lambda n: (n, 0, 0, 0)),
            pl.BlockSpec((2 * V, K * 2 * V), lambda n: (0, 0)),
            pl.BlockSpec((O, (K + 1) * C), lambda n: (0, 0)),
            pl.BlockSpec((1, O), lambda n: (0, 0)),
        ],
        out_specs=pl.BlockSpec((1, T, V, O), lambda n: (n, 0, 0, 0)),
        compiler_params=pltpu.CompilerParams(
            dimension_semantics=("parallel",)),
    )(x_nctv, A2cat, W_all, shift)

    return out_tvo.transpose(0, 3, 1, 2)


# speedup vs baseline: 3.2494x; 1.2281x over previous
import jax
import jax.numpy as jnp
from jax.experimental import pallas as pl
from jax.experimental.pallas import tpu as pltpu


def _make_fused_kernel(C, T, V, K, NB):
    def _fused_kernel(x_ref, a_ref, w_ref, shift_ref, o_ref):
        O = w_ref.shape[0]
        xe = x_ref[:, :, 0::2, :].reshape(NB * C * T // 2, V)
        xo = x_ref[:, :, 1::2, :].reshape(NB * C * T // 2, V)
        xp = jnp.concatenate([xe, xo], axis=1).astype(jnp.bfloat16)
        cat = jnp.dot(xp, a_ref[...],
                      preferred_element_type=jnp.float32).astype(jnp.bfloat16)
        R = C * T // 2
        panels = []
        for n in range(NB):
            parts = [cat[n * R:(n + 1) * R,
                         2 * V * k:2 * V * (k + 1)].reshape(C, T * V)
                     for k in range(K)]
            parts.append(xp[n * R:(n + 1) * R].reshape(C, T * V))
            panels.append(jnp.concatenate(parts, axis=0))
        big = jnp.concatenate(panels, axis=1)
        yT = jax.lax.dot_general(big, w_ref[...], (((0,), (1,)), ((), ())),
                                 preferred_element_type=jnp.float32)
        yT = jnp.maximum(yT + shift_ref[...], 0.0)
        o_ref[...] = yT.reshape(NB, T, V, O)
    return _fused_kernel


@jax.jit
def kernel(x_nctv, A, W, b, bn_gamma, bn_beta, bn_mean, bn_var,
           Wd, bd, dbn_gamma, dbn_beta, dbn_mean, dbn_var, eps=1e-5):
    N, C, T, V = x_nctv.shape
    K, O, _ = W.shape

    An = A / (jnp.sqrt(jnp.sum(A * A, axis=1, keepdims=True)) + 1e-4)
    bn_scale = bn_gamma / jnp.sqrt(bn_var + eps)
    bn_shift = bn_beta - bn_mean * bn_scale
    d_scale = dbn_gamma / jnp.sqrt(dbn_var + eps)
    d_shift = dbn_beta - dbn_mean * d_scale

    W_fold = W * bn_scale[None, :, None]
    Wd_fold = Wd * d_scale[:, None]
    W_all = jnp.concatenate([W_fold[0], W_fold[1], W_fold[2], Wd_fold],
                            axis=1).astype(jnp.bfloat16)
    shift = (bn_scale * jnp.sum(b, axis=0) + bn_shift
             + d_scale * bd + d_shift).reshape(1, O)

    z = jnp.zeros((K, V, V), jnp.float32)
    A2 = jnp.concatenate([jnp.concatenate([An, z], axis=2),
                          jnp.concatenate([z, An], axis=2)], axis=1)
    A2cat = jnp.concatenate([A2[k] for k in range(K)],
                            axis=1).astype(jnp.bfloat16)

    NB = 2
    out_tvo = pl.pallas_call(
        _make_fused_kernel(C, T, V, K, NB),
        out_shape=jax.ShapeDtypeStruct((N, T, V, O), jnp.float32),
        grid=(N // NB,),
        in_specs=[
            pl.BlockSpec((NB, C, T, V), lambda n: (n, 0, 0, 0)),
            pl.BlockSpec((2 * V, K * 2 * V), lambda n: (0, 0)),
            pl.BlockSpec((O, (K + 1) * C), lambda n: (0, 0)),
            pl.BlockSpec((1, O), lambda n: (0, 0)),
        ],
        out_specs=pl.BlockSpec((NB, T, V, O), lambda n: (n, 0, 0, 0)),
        compiler_params=pltpu.CompilerParams(
            dimension_semantics=("parallel",)),
    )(x_nctv, A2cat, W_all, shift)

    return out_tvo.transpose(0, 3, 1, 2)


# speedup vs baseline: 3.7016x; 1.1392x over previous
import jax
import jax.numpy as jnp
from jax.experimental import pallas as pl
from jax.experimental.pallas import tpu as pltpu


def _make_fused_kernel(C, T, V, K, NB):
    def _fused_kernel(x_ref, a_ref, w_ref, shift_ref, o_ref):
        O = w_ref.shape[0]
        xe = x_ref[:, :, 0::2, :].reshape(NB * C * T // 2, V)
        xo = x_ref[:, :, 1::2, :].reshape(NB * C * T // 2, V)
        xp = jnp.concatenate([xe, xo], axis=1).astype(jnp.bfloat16)
        cat = jnp.dot(xp, a_ref[...],
                      preferred_element_type=jnp.float32).astype(jnp.bfloat16)
        R = C * T // 2
        panels = []
        for n in range(NB):
            parts = [cat[n * R:(n + 1) * R,
                         2 * V * k:2 * V * (k + 1)].reshape(C, T * V)
                     for k in range(K)]
            parts.append(xp[n * R:(n + 1) * R].reshape(C, T * V))
            panels.append(jnp.concatenate(parts, axis=0))
        big = jnp.concatenate(panels, axis=1)
        yT = jax.lax.dot_general(big, w_ref[...], (((0,), (1,)), ((), ())),
                                 preferred_element_type=jnp.float32)
        yT = jnp.maximum(yT + shift_ref[...], 0.0)
        o_ref[...] = yT.reshape(NB, T, V, O)
    return _fused_kernel


@jax.jit
def kernel(x_nctv, A, W, b, bn_gamma, bn_beta, bn_mean, bn_var,
           Wd, bd, dbn_gamma, dbn_beta, dbn_mean, dbn_var, eps=1e-5):
    N, C, T, V = x_nctv.shape
    K, O, _ = W.shape

    An = A / (jnp.sqrt(jnp.sum(A * A, axis=1, keepdims=True)) + 1e-4)
    bn_scale = bn_gamma / jnp.sqrt(bn_var + eps)
    bn_shift = bn_beta - bn_mean * bn_scale
    d_scale = dbn_gamma / jnp.sqrt(dbn_var + eps)
    d_shift = dbn_beta - dbn_mean * d_scale

    W_fold = W * bn_scale[None, :, None]
    Wd_fold = Wd * d_scale[:, None]
    W_all = jnp.concatenate([W_fold[0], W_fold[1], W_fold[2], Wd_fold],
                            axis=1).astype(jnp.bfloat16)
    shift = (bn_scale * jnp.sum(b, axis=0) + bn_shift
             + d_scale * bd + d_shift).reshape(1, O)

    z = jnp.zeros((K, V, V), jnp.float32)
    A2 = jnp.concatenate([jnp.concatenate([An, z], axis=2),
                          jnp.concatenate([z, An], axis=2)], axis=1)
    A2cat = jnp.concatenate([A2[k] for k in range(K)],
                            axis=1).astype(jnp.bfloat16)

    NB = 4
    out_tvo = pl.pallas_call(
        _make_fused_kernel(C, T, V, K, NB),
        out_shape=jax.ShapeDtypeStruct((N, T, V, O), jnp.float32),
        grid=(N // NB,),
        in_specs=[
            pl.BlockSpec((NB, C, T, V), lambda n: (n, 0, 0, 0)),
            pl.BlockSpec((2 * V, K * 2 * V), lambda n: (0, 0)),
            pl.BlockSpec((O, (K + 1) * C), lambda n: (0, 0)),
            pl.BlockSpec((1, O), lambda n: (0, 0)),
        ],
        out_specs=pl.BlockSpec((NB, T, V, O), lambda n: (n, 0, 0, 0)),
        compiler_params=pltpu.CompilerParams(
            dimension_semantics=("parallel",)),
    )(x_nctv, A2cat, W_all, shift)

    return out_tvo.transpose(0, 3, 1, 2)


# speedup vs baseline: 3.7036x; 1.0005x over previous
import jax
import jax.numpy as jnp
from jax.experimental import pallas as pl
from jax.experimental.pallas import tpu as pltpu


def _make_fused_kernel(C, T, V, K, NB):
    def _fused_kernel(x_ref, a_ref, w_ref, shift_ref, o_ref):
        O = w_ref.shape[0]
        xe = x_ref[:, :, 0::2, :].reshape(NB * C * T // 2, V)
        xo = x_ref[:, :, 1::2, :].reshape(NB * C * T // 2, V)
        xp = jnp.concatenate([xe, xo], axis=1).astype(jnp.bfloat16)
        cat = jnp.dot(xp, a_ref[...],
                      preferred_element_type=jnp.float32).astype(jnp.bfloat16)
        R = C * T // 2
        panels = []
        for n in range(NB):
            parts = [cat[n * R:(n + 1) * R,
                         2 * V * k:2 * V * (k + 1)].reshape(C, T * V)
                     for k in range(K)]
            parts.append(xp[n * R:(n + 1) * R].reshape(C, T * V))
            panels.append(jnp.concatenate(parts, axis=0))
        big = jnp.concatenate(panels, axis=1)
        yT = jax.lax.dot_general(big, w_ref[...], (((0,), (1,)), ((), ())),
                                 preferred_element_type=jnp.float32)
        yT = jnp.maximum(yT + shift_ref[...], 0.0)
        o_ref[...] = yT.reshape(NB, T, V, O)
    return _fused_kernel


@jax.jit
def kernel(x_nctv, A, W, b, bn_gamma, bn_beta, bn_mean, bn_var,
           Wd, bd, dbn_gamma, dbn_beta, dbn_mean, dbn_var, eps=1e-5):
    N, C, T, V = x_nctv.shape
    K, O, _ = W.shape

    An = A / (jnp.sqrt(jnp.sum(A * A, axis=1, keepdims=True)) + 1e-4)
    bn_scale = bn_gamma / jnp.sqrt(bn_var + eps)
    bn_shift = bn_beta - bn_mean * bn_scale
    d_scale = dbn_gamma / jnp.sqrt(dbn_var + eps)
    d_shift = dbn_beta - dbn_mean * d_scale

    W_fold = W * bn_scale[None, :, None]
    Wd_fold = Wd * d_scale[:, None]
    W_all = jnp.concatenate([W_fold[0], W_fold[1], W_fold[2], Wd_fold],
                            axis=1).astype(jnp.bfloat16)
    shift = (bn_scale * jnp.sum(b, axis=0) + bn_shift
             + d_scale * bd + d_shift).reshape(1, O)

    z = jnp.zeros((K, V, V), jnp.float32)
    A2 = jnp.concatenate([jnp.concatenate([An, z], axis=2),
                          jnp.concatenate([z, An], axis=2)], axis=1)
    A2cat = jnp.concatenate([A2[k] for k in range(K)],
                            axis=1).astype(jnp.bfloat16)

    NB = 4 if N % 4 == 0 else 1
    out_tvo = pl.pallas_call(
        _make_fused_kernel(C, T, V, K, NB),
        out_shape=jax.ShapeDtypeStruct((N, T, V, O), jnp.float32),
        grid=(N // NB,),
        in_specs=[
            pl.BlockSpec((NB, C, T, V), lambda n: (n, 0, 0, 0)),
            pl.BlockSpec((2 * V, K * 2 * V), lambda n: (0, 0)),
            pl.BlockSpec((O, (K + 1) * C), lambda n: (0, 0)),
            pl.BlockSpec((1, O), lambda n: (0, 0)),
        ],
        out_specs=pl.BlockSpec((NB, T, V, O), lambda n: (n, 0, 0, 0)),
        compiler_params=pltpu.CompilerParams(
            dimension_semantics=("parallel",)),
    )(x_nctv, A2cat, W_all, shift)

    return out_tvo.transpose(0, 3, 1, 2)
